# Initial kernel scaffold; baseline (speedup 1.0000x reference)
import jax
import jax.numpy as jnp
from jax.experimental import pallas as pl


def kernel(x, edge_index, W0, W1, W2):
    raise NotImplementedError("write your pallas kernel here")


# speedup vs baseline: 2.6193x; 2.6193x over previous
import functools

import jax
import jax.numpy as jnp
from jax import lax
from jax.experimental import pallas as pl
from jax.experimental.pallas import tpu as pltpu
from jax.experimental.pallas import tpu_sc as plsc

N = 10000
E = 160000
D = 256
DEPTH = 3
H = 128

NC = 2
NS = 16
K = 128
CHUNKS = 80
EP = NS * CHUNKS * K
IDX_ROWS = EP // K
ACC_ROWS = 10240
ZERO_PER_TILE = ACC_ROWS // NS
OUT_PER_TILE = 1000


def _seg_sum_sc(h_stacked, src0, src1, dst2, zeros):
  mesh = plsc.VectorSubcoreMesh(core_axis_name="c", subcore_axis_name="s",
                                num_cores=NC, num_subcores=NS)

  @functools.partial(
      pl.kernel,
      mesh=mesh,
      out_type=jax.ShapeDtypeStruct((2 * N, H), jnp.float32),
      scratch_types=[
          pltpu.VMEM((IDX_ROWS // NS, K), jnp.int32),
          pltpu.VMEM((IDX_ROWS // NS, K), jnp.int32),
          pltpu.VMEM((K, H), jnp.float32),
          pltpu.VMEM_SHARED((ACC_ROWS, H), jnp.float32),
      ],
  )
  def k(h_hbm, src0_hbm, src1_hbm, dst_hbm, zeros_hbm, out_hbm,
        src_v, dst_v, rows_v, acc):
    cid = lax.axis_index("c")
    sid = lax.axis_index("s")
    rows_per_tile = IDX_ROWS // NS

    @pl.when(cid == 0)
    def _():
      pltpu.sync_copy(src0_hbm.at[pl.ds(sid * rows_per_tile, rows_per_tile)],
                      src_v)

    @pl.when(cid != 0)
    def _():
      pltpu.sync_copy(src1_hbm.at[pl.ds(sid * rows_per_tile, rows_per_tile)],
                      src_v)

    pltpu.sync_copy(dst_hbm.at[pl.ds(sid * rows_per_tile, rows_per_tile)],
                    dst_v)

    pltpu.sync_copy(zeros_hbm.at[pl.ds(sid * ZERO_PER_TILE, ZERO_PER_TILE)],
                    acc.at[pl.ds(sid * ZERO_PER_TILE, ZERO_PER_TILE)])
    plsc.subcore_barrier()

    def body(j, carry):
      pltpu.sync_copy(h_hbm.at[src_v.at[j]], rows_v)
      pltpu.sync_copy(rows_v, acc.at[dst_v.at[j]], add=True)
      return carry

    lax.fori_loop(0, rows_per_tile, body, 0)
    plsc.subcore_barrier()

    @pl.when(sid < 10)
    def _():
      pltpu.sync_copy(
          acc.at[pl.ds(sid * OUT_PER_TILE, OUT_PER_TILE)],
          out_hbm.at[pl.ds(cid * N + sid * OUT_PER_TILE, OUT_PER_TILE)])

  return k(h_stacked, src0, src1, dst2, zeros)


RB = 2000
NB = N // RB


def _mm_h_kernel(x_ref, w_ref, o_ref):
  o_ref[...] = jnp.dot(x_ref[...], w_ref[...],
                       preferred_element_type=jnp.float32)


def _mm_h(x, w0):
  return pl.pallas_call(
      _mm_h_kernel,
      grid=(NC, NB),
      in_specs=[
          pl.BlockSpec((RB, D), lambda c, i: (i, 0)),
          pl.BlockSpec((D, H), lambda c, i: (0, c)),
      ],
      out_specs=pl.BlockSpec((RB, H), lambda c, i: (c * NB + i, 0)),
      out_shape=jax.ShapeDtypeStruct((2 * N, H), jnp.float32),
  )(x, w0)


def _combine_kernel(x_ref, ma_ref, mb_ref, wf_ref, wa_ref, wb_ref, o_ref):
  y = jnp.dot(x_ref[...], wf_ref[...], preferred_element_type=jnp.float32)
  y += jnp.dot(ma_ref[...], wa_ref[...], preferred_element_type=jnp.float32)
  y += jnp.dot(mb_ref[...], wb_ref[...], preferred_element_type=jnp.float32)
  mu = jnp.mean(y, axis=1, keepdims=True)
  d = y - mu
  var = jnp.mean(d * d, axis=1, keepdims=True)
  o_ref[...] = y * lax.rsqrt(var)


def _combine(x, msg, wf, w2b0, w2b1):
  return pl.pallas_call(
      _combine_kernel,
      grid=(NB,),
      in_specs=[
          pl.BlockSpec((RB, D), lambda i: (i, 0)),
          pl.BlockSpec((RB, H), lambda i: (i, 0)),
          pl.BlockSpec((RB, H), lambda i: (i + NB, 0)),
          pl.BlockSpec((D, D), lambda i: (0, 0)),
          pl.BlockSpec((H, D), lambda i: (0, 0)),
          pl.BlockSpec((H, D), lambda i: (0, 0)),
      ],
      out_specs=pl.BlockSpec((RB, D), lambda i: (i, 0)),
      out_shape=jax.ShapeDtypeStruct((N, D), jnp.float32),
  )(x, msg, msg, wf, w2b0, w2b1)


def _prep_kernel(w1_ref, w2_ref, o_ref):
  o_ref[0] = jnp.dot(w1_ref[0], w2_ref[0],
                     preferred_element_type=jnp.float32)


def _prep(w1, w2a):
  return pl.pallas_call(
      _prep_kernel,
      grid=(DEPTH,),
      in_specs=[
          pl.BlockSpec((1, D, D), lambda i: (i, 0, 0)),
          pl.BlockSpec((1, D, D), lambda i: (i, 0, 0)),
      ],
      out_specs=pl.BlockSpec((1, D, D), lambda i: (i, 0, 0)),
      out_shape=jax.ShapeDtypeStruct((DEPTH, D, D), jnp.float32),
  )(w1, w2a)


def kernel(x, edge_index, W0, W1, W2):
  src = edge_index[0].astype(jnp.int32)
  dst = edge_index[1].astype(jnp.int32)
  pad = EP - E
  src_p = jnp.concatenate([src, jnp.zeros((pad,), jnp.int32)])
  dst_p = jnp.concatenate([dst, jnp.full((pad,), N, jnp.int32)])
  src0 = src_p.reshape(IDX_ROWS, K)
  src1 = src0 + N
  dst2 = dst_p.reshape(IDX_ROWS, K)
  zeros = jnp.zeros((ACC_ROWS, H), jnp.float32)

  wf = _prep(W1, W2[:, :D, :])
  w2b0 = W2[:, D:D + H, :]
  w2b1 = W2[:, D + H:, :]

  for i in range(DEPTH):
    h_stacked = _mm_h(x, W0[i])
    msg = _seg_sum_sc(h_stacked, src0, src1, dst2, zeros)
    x = _combine(x, msg, wf[i], w2b0[i], w2b1[i])
  return x


# speedup vs baseline: 2.7119x; 1.0353x over previous
import functools

import jax
import jax.numpy as jnp
from jax import lax
from jax.experimental import pallas as pl
from jax.experimental.pallas import tpu as pltpu
from jax.experimental.pallas import tpu_sc as plsc

N = 10000
E = 160000
D = 256
DEPTH = 3
H = 128

NC = 2
NS = 16
K = 64
CHUNKS = 160
EP = NS * CHUNKS * K
IDX_ROWS = EP // K
ACC_ROWS = 10240
ZERO_PER_TILE = ACC_ROWS // NS
OUT_PER_TILE = 1000


def _seg_sum_sc(h_stacked, src0, src1, dst2, zeros):
  mesh = plsc.VectorSubcoreMesh(core_axis_name="c", subcore_axis_name="s",
                                num_cores=NC, num_subcores=NS)

  @functools.partial(
      pl.kernel,
      mesh=mesh,
      out_type=jax.ShapeDtypeStruct((2 * N, H), jnp.float32),
      scratch_types=[
          pltpu.VMEM((CHUNKS // 2, K), jnp.int32),
          pltpu.VMEM((CHUNKS // 2, K), jnp.int32),
          pltpu.VMEM((K, H), jnp.float32),
          pltpu.VMEM((K, H), jnp.float32),
          pltpu.VMEM_SHARED((ACC_ROWS, H), jnp.float32),
          pltpu.SemaphoreType.DMA,
          pltpu.SemaphoreType.DMA,
      ],
  )
  def k(h_hbm, src0_hbm, src1_hbm, dst_hbm, zeros_hbm, out_hbm,
        src_v, dst_v, rows0_v, rows1_v, acc, sem0, sem1):
    cid = lax.axis_index("c")
    sid = lax.axis_index("s")
    rows_per_tile = CHUNKS
    hr = CHUNKS // 2

    pltpu.sync_copy(zeros_hbm.at[pl.ds(sid * ZERO_PER_TILE, ZERO_PER_TILE)],
                    acc.at[pl.ds(sid * ZERO_PER_TILE, ZERO_PER_TILE)])
    plsc.subcore_barrier()

    def gather(j, buf, sem):
      return pltpu.async_copy(h_hbm.at[src_v.at[j]], buf, sem)

    for half in range(2):
      base = sid * rows_per_tile + half * hr
      @pl.when(cid == 0)
      def _():
        pltpu.sync_copy(src0_hbm.at[pl.ds(base, hr)], src_v)

      @pl.when(cid != 0)
      def _():
        pltpu.sync_copy(src1_hbm.at[pl.ds(base, hr)], src_v)

      pltpu.sync_copy(dst_hbm.at[pl.ds(base, hr)], dst_v)

      gather(0, rows0_v, sem0)

      def body(t, carry):
        j0 = 2 * t
        j1 = j0 + 1
        pltpu.make_async_copy(h_hbm.at[src_v.at[j0]], rows0_v, sem0).wait()
        gather(j1, rows1_v, sem1)
        pltpu.sync_copy(rows0_v, acc.at[dst_v.at[j0]], add=True)
        pltpu.make_async_copy(h_hbm.at[src_v.at[j1]], rows1_v, sem1).wait()

        @pl.when(j1 + 1 < hr)
        def _():
          gather(j1 + 1, rows0_v, sem0)

        pltpu.sync_copy(rows1_v, acc.at[dst_v.at[j1]], add=True)
        return carry

      lax.fori_loop(0, hr // 2, body, 0)
    plsc.subcore_barrier()

    @pl.when(sid < 10)
    def _():
      pltpu.sync_copy(
          acc.at[pl.ds(sid * OUT_PER_TILE, OUT_PER_TILE)],
          out_hbm.at[pl.ds(cid * N + sid * OUT_PER_TILE, OUT_PER_TILE)])

  return k(h_stacked, src0, src1, dst2, zeros)


RB = 2000
NB = N // RB


def _mm_h_kernel(x_ref, w_ref, o_ref):
  o_ref[...] = jnp.dot(x_ref[...], w_ref[...],
                       preferred_element_type=jnp.float32)


def _mm_h(x, w0):
  return pl.pallas_call(
      _mm_h_kernel,
      grid=(NC, NB),
      in_specs=[
          pl.BlockSpec((RB, D), lambda c, i: (i, 0)),
          pl.BlockSpec((D, H), lambda c, i: (0, c)),
      ],
      out_specs=pl.BlockSpec((RB, H), lambda c, i: (c * NB + i, 0)),
      out_shape=jax.ShapeDtypeStruct((2 * N, H), jnp.float32),
  )(x, w0)


def _combine_kernel(x_ref, ma_ref, mb_ref, wf_ref, wa_ref, wb_ref, o_ref):
  y = jnp.dot(x_ref[...], wf_ref[...], preferred_element_type=jnp.float32)
  y += jnp.dot(ma_ref[...], wa_ref[...], preferred_element_type=jnp.float32)
  y += jnp.dot(mb_ref[...], wb_ref[...], preferred_element_type=jnp.float32)
  mu = jnp.mean(y, axis=1, keepdims=True)
  d = y - mu
  var = jnp.mean(d * d, axis=1, keepdims=True)
  o_ref[...] = y * lax.rsqrt(var)


def _combine(x, msg, wf, w2b0, w2b1):
  return pl.pallas_call(
      _combine_kernel,
      grid=(NB,),
      in_specs=[
          pl.BlockSpec((RB, D), lambda i: (i, 0)),
          pl.BlockSpec((RB, H), lambda i: (i, 0)),
          pl.BlockSpec((RB, H), lambda i: (i + NB, 0)),
          pl.BlockSpec((D, D), lambda i: (0, 0)),
          pl.BlockSpec((H, D), lambda i: (0, 0)),
          pl.BlockSpec((H, D), lambda i: (0, 0)),
      ],
      out_specs=pl.BlockSpec((RB, D), lambda i: (i, 0)),
      out_shape=jax.ShapeDtypeStruct((N, D), jnp.float32),
  )(x, msg, msg, wf, w2b0, w2b1)


def _prep_kernel(w1_ref, w2_ref, o_ref):
  o_ref[0] = jnp.dot(w1_ref[0], w2_ref[0],
                     preferred_element_type=jnp.float32)


def _prep(w1, w2a):
  return pl.pallas_call(
      _prep_kernel,
      grid=(DEPTH,),
      in_specs=[
          pl.BlockSpec((1, D, D), lambda i: (i, 0, 0)),
          pl.BlockSpec((1, D, D), lambda i: (i, 0, 0)),
      ],
      out_specs=pl.BlockSpec((1, D, D), lambda i: (i, 0, 0)),
      out_shape=jax.ShapeDtypeStruct((DEPTH, D, D), jnp.float32),
  )(w1, w2a)


def kernel(x, edge_index, W0, W1, W2):
  src = edge_index[0].astype(jnp.int32)
  dst = edge_index[1].astype(jnp.int32)
  pad = EP - E
  src_p = jnp.concatenate([src, jnp.zeros((pad,), jnp.int32)])
  dst_p = jnp.concatenate([dst, jnp.full((pad,), N, jnp.int32)])
  src0 = src_p.reshape(IDX_ROWS, K)
  src1 = src0 + N
  dst2 = dst_p.reshape(IDX_ROWS, K)
  zeros = jnp.zeros((ACC_ROWS, H), jnp.float32)

  wf = _prep(W1, W2[:, :D, :])
  w2b0 = W2[:, D:D + H, :]
  w2b1 = W2[:, D + H:, :]

  for i in range(DEPTH):
    h_stacked = _mm_h(x, W0[i])
    msg = _seg_sum_sc(h_stacked, src0, src1, dst2, zeros)
    x = _combine(x, msg, wf[i], w2b0[i], w2b1[i])
  return x


# speedup vs baseline: 2.7552x; 1.0160x over previous
import functools

import jax
import jax.numpy as jnp
from jax import lax
from jax.experimental import pallas as pl
from jax.experimental.pallas import tpu as pltpu
from jax.experimental.pallas import tpu_sc as plsc

N = 10000
E = 160000
D = 256
DEPTH = 3
H = 128

NC = 2
NS = 16
K = 64
CHUNKS = 160
EP = NS * CHUNKS * K
IDX_ROWS = EP // K
ACC_ROWS = 10240
ZERO_PER_TILE = ACC_ROWS // NS
OUT_PER_TILE = 1000


def _seg_sum_sc(h_stacked, src0, src1, dst2, zeros):
  mesh = plsc.VectorSubcoreMesh(core_axis_name="c", subcore_axis_name="s",
                                num_cores=NC, num_subcores=NS)

  @functools.partial(
      pl.kernel,
      mesh=mesh,
      out_type=jax.ShapeDtypeStruct((2 * N, H), jnp.float32),
      scratch_types=[
          pltpu.VMEM((CHUNKS // 2, K), jnp.int32),
          pltpu.VMEM((CHUNKS // 2, K), jnp.int32),
          pltpu.VMEM((K, H), jnp.float32),
          pltpu.VMEM((K, H), jnp.float32),
          pltpu.VMEM_SHARED((ACC_ROWS, H), jnp.float32),
          pltpu.SemaphoreType.DMA,
          pltpu.SemaphoreType.DMA,
          pltpu.SemaphoreType.DMA,
          pltpu.SemaphoreType.DMA,
      ],
  )
  def k(h_hbm, src0_hbm, src1_hbm, dst_hbm, zeros_hbm, out_hbm,
        src_v, dst_v, rows0_v, rows1_v, acc, g0, g1, s0, s1):
    cid = lax.axis_index("c")
    sid = lax.axis_index("s")
    rows_per_tile = CHUNKS
    hr = CHUNKS // 2

    pltpu.sync_copy(zeros_hbm.at[pl.ds(sid * ZERO_PER_TILE, ZERO_PER_TILE)],
                    acc.at[pl.ds(sid * ZERO_PER_TILE, ZERO_PER_TILE)])
    plsc.subcore_barrier()

    def gather(j, buf, sem):
      return pltpu.async_copy(h_hbm.at[src_v.at[j]], buf, sem)

    for half in range(2):
      base = sid * rows_per_tile + half * hr
      @pl.when(cid == 0)
      def _():
        pltpu.sync_copy(src0_hbm.at[pl.ds(base, hr)], src_v)

      @pl.when(cid != 0)
      def _():
        pltpu.sync_copy(src1_hbm.at[pl.ds(base, hr)], src_v)

      pltpu.sync_copy(dst_hbm.at[pl.ds(base, hr)], dst_v)

      gather(0, rows0_v, g0)
      gather(1, rows1_v, g1)

      def wait_gather(j, buf, sem):
        pltpu.make_async_copy(h_hbm.at[src_v.at[j]], buf, sem).wait()

      def scatter(j, buf, sem):
        return pltpu.async_copy(buf, acc.at[dst_v.at[j]], sem, add=True)

      def wait_scatter(j, buf, sem):
        pltpu.make_async_copy(buf, acc.at[dst_v.at[j]], sem).wait()

      def body(t, carry):
        j0 = 2 * t
        j1 = j0 + 1
        wait_gather(j0, rows0_v, g0)
        scatter(j0, rows0_v, s0)
        wait_gather(j1, rows1_v, g1)
        scatter(j1, rows1_v, s1)

        @pl.when(j0 + 2 < hr)
        def _():
          wait_scatter(j0, rows0_v, s0)
          gather(j0 + 2, rows0_v, g0)

        @pl.when(j1 + 2 < hr)
        def _():
          wait_scatter(j1, rows1_v, s1)
          gather(j1 + 2, rows1_v, g1)

        return carry

      lax.fori_loop(0, hr // 2, body, 0)
      wait_scatter(hr - 2, rows0_v, s0)
      wait_scatter(hr - 1, rows1_v, s1)
    plsc.subcore_barrier()

    @pl.when(sid < 10)
    def _():
      pltpu.sync_copy(
          acc.at[pl.ds(sid * OUT_PER_TILE, OUT_PER_TILE)],
          out_hbm.at[pl.ds(cid * N + sid * OUT_PER_TILE, OUT_PER_TILE)])

  return k(h_stacked, src0, src1, dst2, zeros)


RB = 2000
NB = N // RB


def _mm_h_kernel(x_ref, w_ref, o_ref):
  o_ref[...] = jnp.dot(x_ref[...], w_ref[...],
                       preferred_element_type=jnp.float32)


def _mm_h(x, w0):
  return pl.pallas_call(
      _mm_h_kernel,
      grid=(NC, NB),
      in_specs=[
          pl.BlockSpec((RB, D), lambda c, i: (i, 0)),
          pl.BlockSpec((D, H), lambda c, i: (0, c)),
      ],
      out_specs=pl.BlockSpec((RB, H), lambda c, i: (c * NB + i, 0)),
      out_shape=jax.ShapeDtypeStruct((2 * N, H), jnp.float32),
  )(x, w0)


def _combine_kernel(x_ref, ma_ref, mb_ref, wf_ref, wa_ref, wb_ref, o_ref):
  y = jnp.dot(x_ref[...], wf_ref[...], preferred_element_type=jnp.float32)
  y += jnp.dot(ma_ref[...], wa_ref[...], preferred_element_type=jnp.float32)
  y += jnp.dot(mb_ref[...], wb_ref[...], preferred_element_type=jnp.float32)
  mu = jnp.mean(y, axis=1, keepdims=True)
  d = y - mu
  var = jnp.mean(d * d, axis=1, keepdims=True)
  o_ref[...] = y * lax.rsqrt(var)


def _combine(x, msg, wf, w2b0, w2b1):
  return pl.pallas_call(
      _combine_kernel,
      grid=(NB,),
      in_specs=[
          pl.BlockSpec((RB, D), lambda i: (i, 0)),
          pl.BlockSpec((RB, H), lambda i: (i, 0)),
          pl.BlockSpec((RB, H), lambda i: (i + NB, 0)),
          pl.BlockSpec((D, D), lambda i: (0, 0)),
          pl.BlockSpec((H, D), lambda i: (0, 0)),
          pl.BlockSpec((H, D), lambda i: (0, 0)),
      ],
      out_specs=pl.BlockSpec((RB, D), lambda i: (i, 0)),
      out_shape=jax.ShapeDtypeStruct((N, D), jnp.float32),
  )(x, msg, msg, wf, w2b0, w2b1)


def _prep_kernel(w1_ref, w2_ref, o_ref):
  o_ref[0] = jnp.dot(w1_ref[0], w2_ref[0],
                     preferred_element_type=jnp.float32)


def _prep(w1, w2a):
  return pl.pallas_call(
      _prep_kernel,
      grid=(DEPTH,),
      in_specs=[
          pl.BlockSpec((1, D, D), lambda i: (i, 0, 0)),
          pl.BlockSpec((1, D, D), lambda i: (i, 0, 0)),
      ],
      out_specs=pl.BlockSpec((1, D, D), lambda i: (i, 0, 0)),
      out_shape=jax.ShapeDtypeStruct((DEPTH, D, D), jnp.float32),
  )(w1, w2a)


def kernel(x, edge_index, W0, W1, W2):
  src = edge_index[0].astype(jnp.int32)
  dst = edge_index[1].astype(jnp.int32)
  pad = EP - E
  src_p = jnp.concatenate([src, jnp.zeros((pad,), jnp.int32)])
  dst_p = jnp.concatenate([dst, jnp.full((pad,), N, jnp.int32)])
  src0 = src_p.reshape(IDX_ROWS, K)
  src1 = src0 + N
  dst2 = dst_p.reshape(IDX_ROWS, K)
  zeros = jnp.zeros((ACC_ROWS, H), jnp.float32)

  wf = _prep(W1, W2[:, :D, :])
  w2b0 = W2[:, D:D + H, :]
  w2b1 = W2[:, D + H:, :]

  for i in range(DEPTH):
    h_stacked = _mm_h(x, W0[i])
    msg = _seg_sum_sc(h_stacked, src0, src1, dst2, zeros)
    x = _combine(x, msg, wf[i], w2b0[i], w2b1[i])
  return x


# speedup vs baseline: 5.8271x; 2.1150x over previous
import functools

import jax
import jax.numpy as jnp
from jax import lax
from jax.experimental import pallas as pl
from jax.experimental.pallas import tpu as pltpu
from jax.experimental.pallas import tpu_sc as plsc

N = 10000
E = 160000
D = 256
DEPTH = 3
H = 128

NC = 2
NS = 16
K = 128
CHUNKS = 80
PIPE = 2
HALVES = 2
EP = NS * CHUNKS * K
IDX_ROWS = EP // K
ACC_ROWS = 10240
ZERO_PER_TILE = ACC_ROWS // NS
OUT_PER_TILE = 1000


def _seg_sum_sc(h_stacked, src0, src1, dst2, zeros):
  mesh = plsc.VectorSubcoreMesh(core_axis_name="c", subcore_axis_name="s",
                                num_cores=NC, num_subcores=NS)

  @functools.partial(
      pl.kernel,
      mesh=mesh,
      out_type=jax.ShapeDtypeStruct((2 * N, H), jnp.float32),
      scratch_types=[
          pltpu.VMEM((CHUNKS // HALVES, K), jnp.int32),
          pltpu.VMEM((CHUNKS // HALVES, K), jnp.int32),
      ] + [pltpu.VMEM((K, H), jnp.float32)] * PIPE
      + [pltpu.VMEM_SHARED((ACC_ROWS, H), jnp.float32)]
      + [pltpu.SemaphoreType.DMA] * (2 * PIPE),
  )
  def k(h_hbm, src0_hbm, src1_hbm, dst_hbm, zeros_hbm, out_hbm,
        src_v, dst_v, *rest):
    rows = rest[:PIPE]
    acc = rest[PIPE]
    gsem = rest[PIPE + 1:2 * PIPE + 1]
    ssem = rest[2 * PIPE + 1:]
    cid = lax.axis_index("c")
    sid = lax.axis_index("s")
    hr = CHUNKS // HALVES

    pltpu.sync_copy(zeros_hbm.at[pl.ds(sid * ZERO_PER_TILE, ZERO_PER_TILE)],
                    acc.at[pl.ds(sid * ZERO_PER_TILE, ZERO_PER_TILE)])
    plsc.subcore_barrier()

    def gather(j, b):
      pltpu.async_copy(h_hbm.at[src_v.at[j]], rows[b], gsem[b])

    def wait_gather(j, b):
      pltpu.make_async_copy(h_hbm.at[src_v.at[j]], rows[b], gsem[b]).wait()

    def scatter(j, b):
      pltpu.async_copy(rows[b], acc.at[dst_v.at[j]], ssem[b], add=True)

    def wait_scatter(j, b):
      pltpu.make_async_copy(rows[b], acc.at[dst_v.at[j]], ssem[b]).wait()

    for half in range(HALVES):
      base = sid * CHUNKS + half * hr
      @pl.when(cid == 0)
      def _():
        pltpu.sync_copy(src0_hbm.at[pl.ds(base, hr)], src_v)

      @pl.when(cid != 0)
      def _():
        pltpu.sync_copy(src1_hbm.at[pl.ds(base, hr)], src_v)

      pltpu.sync_copy(dst_hbm.at[pl.ds(base, hr)], dst_v)

      for b in range(PIPE):
        gather(b, b)

      def body(t, carry):
        j0 = PIPE * t
        for b in range(PIPE):
          wait_gather(j0 + b, b)
          scatter(j0 + b, b)
        for b in range(PIPE):
          @pl.when(j0 + b + PIPE < hr)
          def _(b=b, j0=j0):
            wait_scatter(j0 + b, b)
            gather(j0 + b + PIPE, b)
        return carry

      lax.fori_loop(0, hr // PIPE, body, 0)
      for b in range(PIPE):
        wait_scatter(hr - PIPE + b, b)
    plsc.subcore_barrier()

    @pl.when(sid < 10)
    def _():
      pltpu.sync_copy(
          acc.at[pl.ds(sid * OUT_PER_TILE, OUT_PER_TILE)],
          out_hbm.at[pl.ds(cid * N + sid * OUT_PER_TILE, OUT_PER_TILE)])

  return k(h_stacked, src0, src1, dst2, zeros)


RB = 2000
NB = N // RB


def _mm_h_kernel(x_ref, w_ref, o_ref):
  o_ref[...] = jnp.dot(x_ref[...], w_ref[...],
                       preferred_element_type=jnp.float32)


def _mm_h(x, w0):
  return pl.pallas_call(
      _mm_h_kernel,
      grid=(NC, NB),
      in_specs=[
          pl.BlockSpec((RB, D), lambda c, i: (i, 0)),
          pl.BlockSpec((D, H), lambda c, i: (0, c)),
      ],
      out_specs=pl.BlockSpec((RB, H), lambda c, i: (c * NB + i, 0)),
      out_shape=jax.ShapeDtypeStruct((2 * N, H), jnp.float32),
  )(x, w0)


def _combine_kernel(x_ref, ma_ref, mb_ref, wf_ref, wa_ref, wb_ref, o_ref):
  y = jnp.dot(x_ref[...], wf_ref[...], preferred_element_type=jnp.float32)
  y += jnp.dot(ma_ref[...], wa_ref[...], preferred_element_type=jnp.float32)
  y += jnp.dot(mb_ref[...], wb_ref[...], preferred_element_type=jnp.float32)
  mu = jnp.mean(y, axis=1, keepdims=True)
  d = y - mu
  var = jnp.mean(d * d, axis=1, keepdims=True)
  o_ref[...] = y * lax.rsqrt(var)


def _combine(x, msg, wf, w2b0, w2b1):
  return pl.pallas_call(
      _combine_kernel,
      grid=(NB,),
      in_specs=[
          pl.BlockSpec((RB, D), lambda i: (i, 0)),
          pl.BlockSpec((RB, H), lambda i: (i, 0)),
          pl.BlockSpec((RB, H), lambda i: (i + NB, 0)),
          pl.BlockSpec((D, D), lambda i: (0, 0)),
          pl.BlockSpec((H, D), lambda i: (0, 0)),
          pl.BlockSpec((H, D), lambda i: (0, 0)),
      ],
      out_specs=pl.BlockSpec((RB, D), lambda i: (i, 0)),
      out_shape=jax.ShapeDtypeStruct((N, D), jnp.float32),
  )(x, msg, msg, wf, w2b0, w2b1)


def _prep_kernel(w1_ref, w2_ref, o_ref):
  o_ref[0] = jnp.dot(w1_ref[0], w2_ref[0],
                     preferred_element_type=jnp.float32)


def _prep(w1, w2a):
  return pl.pallas_call(
      _prep_kernel,
      grid=(DEPTH,),
      in_specs=[
          pl.BlockSpec((1, D, D), lambda i: (i, 0, 0)),
          pl.BlockSpec((1, D, D), lambda i: (i, 0, 0)),
      ],
      out_specs=pl.BlockSpec((1, D, D), lambda i: (i, 0, 0)),
      out_shape=jax.ShapeDtypeStruct((DEPTH, D, D), jnp.float32),
  )(w1, w2a)


def kernel(x, edge_index, W0, W1, W2):
  src = edge_index[0].astype(jnp.int32)
  dst = edge_index[1].astype(jnp.int32)
  pad = EP - E
  pad_ar = jnp.arange(pad, dtype=jnp.int32)
  src_p = jnp.concatenate([src, (pad_ar * 61) % N])
  dst_p = jnp.concatenate([dst, N + pad_ar % (ACC_ROWS - N)])
  src0 = src_p.reshape(IDX_ROWS, K)
  src1 = src0 + N
  dst2 = dst_p.reshape(IDX_ROWS, K)
  zeros = jnp.zeros((ACC_ROWS, H), jnp.float32)

  wf = _prep(W1, W2[:, :D, :])
  w2b0 = W2[:, D:D + H, :]
  w2b1 = W2[:, D + H:, :]

  for i in range(DEPTH):
    h_stacked = _mm_h(x, W0[i])
    msg = _seg_sum_sc(h_stacked, src0, src1, dst2, zeros)
    x = _combine(x, msg, wf[i], w2b0[i], w2b1[i])
  return x


# speedup vs baseline: 6.3278x; 1.0859x over previous
import functools

import jax
import jax.numpy as jnp
from jax import lax
from jax.experimental import pallas as pl
from jax.experimental.pallas import tpu as pltpu
from jax.experimental.pallas import tpu_sc as plsc

N = 10000
E = 160000
D = 256
DEPTH = 3
H = 128

NC = 2
NS = 16
K = 128
CHUNKS = 80
PIPE = 2
HALVES = 2
EP = NS * CHUNKS * K
IDX_ROWS = EP // K
ACC_ROWS = 10240
ZERO_PER_TILE = ACC_ROWS // NS
OUT_PER_TILE = 1000


def _seg_sum_sc(h_stacked, src0, src1, dst2, zeros):
  mesh = plsc.VectorSubcoreMesh(core_axis_name="c", subcore_axis_name="s",
                                num_cores=NC, num_subcores=NS)

  @functools.partial(
      pl.kernel,
      mesh=mesh,
      out_type=jax.ShapeDtypeStruct((2 * N, H), jnp.float32),
      scratch_types=[
          pltpu.VMEM((CHUNKS // HALVES, K), jnp.int32),
          pltpu.VMEM((CHUNKS // HALVES, K), jnp.int32),
      ] + [pltpu.VMEM((K, H), jnp.float32)] * PIPE
      + [pltpu.VMEM_SHARED((ACC_ROWS, H), jnp.float32)]
      + [pltpu.SemaphoreType.DMA] * (2 * PIPE),
  )
  def k(h_hbm, src0_hbm, src1_hbm, dst_hbm, zeros_hbm, out_hbm,
        src_v, dst_v, *rest):
    rows = rest[:PIPE]
    acc = rest[PIPE]
    gsem = rest[PIPE + 1:2 * PIPE + 1]
    ssem = rest[2 * PIPE + 1:]
    cid = lax.axis_index("c")
    sid = lax.axis_index("s")
    hr = CHUNKS // HALVES

    pltpu.sync_copy(zeros_hbm.at[pl.ds(sid * ZERO_PER_TILE, ZERO_PER_TILE)],
                    acc.at[pl.ds(sid * ZERO_PER_TILE, ZERO_PER_TILE)])
    plsc.subcore_barrier()

    def gather(j, b):
      pltpu.async_copy(h_hbm.at[src_v.at[j]], rows[b], gsem[b])

    def wait_gather(j, b):
      pltpu.make_async_copy(h_hbm.at[src_v.at[j]], rows[b], gsem[b]).wait()

    def scatter(j, b):
      pltpu.async_copy(rows[b], acc.at[dst_v.at[j]], ssem[b], add=True)

    def wait_scatter(j, b):
      pltpu.make_async_copy(rows[b], acc.at[dst_v.at[j]], ssem[b]).wait()

    for half in range(HALVES):
      base = sid * CHUNKS + half * hr
      @pl.when(cid == 0)
      def _():
        pltpu.sync_copy(src0_hbm.at[pl.ds(base, hr)], src_v)

      @pl.when(cid != 0)
      def _():
        pltpu.sync_copy(src1_hbm.at[pl.ds(base, hr)], src_v)

      pltpu.sync_copy(dst_hbm.at[pl.ds(base, hr)], dst_v)

      gather(0, 0)

      def step(j, bcur, bnext):
        wait_gather(j, bcur)
        scatter(j, bcur)

        @pl.when(j >= 1)
        def _():
          wait_scatter(j - 1, bnext)

        @pl.when(j + 1 < hr)
        def _():
          gather(j + 1, bnext)

      def body(t, carry):
        step(2 * t, 0, 1)
        step(2 * t + 1, 1, 0)
        return carry

      lax.fori_loop(0, hr // 2, body, 0)
      wait_scatter(hr - 1, 1)
    plsc.subcore_barrier()

    @pl.when(sid < 10)
    def _():
      pltpu.sync_copy(
          acc.at[pl.ds(sid * OUT_PER_TILE, OUT_PER_TILE)],
          out_hbm.at[pl.ds(cid * N + sid * OUT_PER_TILE, OUT_PER_TILE)])

  return k(h_stacked, src0, src1, dst2, zeros)


RB = 2000
NB = N // RB


def _mm_h_kernel(x_ref, w_ref, o_ref):
  o_ref[...] = jnp.dot(x_ref[...], w_ref[...],
                       preferred_element_type=jnp.float32)


def _mm_h(x, w0):
  return pl.pallas_call(
      _mm_h_kernel,
      grid=(NC, NB),
      in_specs=[
          pl.BlockSpec((RB, D), lambda c, i: (i, 0)),
          pl.BlockSpec((D, H), lambda c, i: (0, c)),
      ],
      out_specs=pl.BlockSpec((RB, H), lambda c, i: (c * NB + i, 0)),
      out_shape=jax.ShapeDtypeStruct((2 * N, H), jnp.float32),
  )(x, w0)


def _combine_kernel(x_ref, ma_ref, mb_ref, wf_ref, wa_ref, wb_ref, o_ref):
  y = jnp.dot(x_ref[...], wf_ref[...], preferred_element_type=jnp.float32)
  y += jnp.dot(ma_ref[...], wa_ref[...], preferred_element_type=jnp.float32)
  y += jnp.dot(mb_ref[...], wb_ref[...], preferred_element_type=jnp.float32)
  mu = jnp.mean(y, axis=1, keepdims=True)
  d = y - mu
  var = jnp.mean(d * d, axis=1, keepdims=True)
  o_ref[...] = y * lax.rsqrt(var)


def _combine(x, msg, wf, w2b0, w2b1):
  return pl.pallas_call(
      _combine_kernel,
      grid=(NB,),
      in_specs=[
          pl.BlockSpec((RB, D), lambda i: (i, 0)),
          pl.BlockSpec((RB, H), lambda i: (i, 0)),
          pl.BlockSpec((RB, H), lambda i: (i + NB, 0)),
          pl.BlockSpec((D, D), lambda i: (0, 0)),
          pl.BlockSpec((H, D), lambda i: (0, 0)),
          pl.BlockSpec((H, D), lambda i: (0, 0)),
      ],
      out_specs=pl.BlockSpec((RB, D), lambda i: (i, 0)),
      out_shape=jax.ShapeDtypeStruct((N, D), jnp.float32),
  )(x, msg, msg, wf, w2b0, w2b1)


def _prep_kernel(w1_ref, w2_ref, o_ref):
  o_ref[0] = jnp.dot(w1_ref[0], w2_ref[0],
                     preferred_element_type=jnp.float32)


def _prep(w1, w2a):
  return pl.pallas_call(
      _prep_kernel,
      grid=(DEPTH,),
      in_specs=[
          pl.BlockSpec((1, D, D), lambda i: (i, 0, 0)),
          pl.BlockSpec((1, D, D), lambda i: (i, 0, 0)),
      ],
      out_specs=pl.BlockSpec((1, D, D), lambda i: (i, 0, 0)),
      out_shape=jax.ShapeDtypeStruct((DEPTH, D, D), jnp.float32),
  )(w1, w2a)


def kernel(x, edge_index, W0, W1, W2):
  src = edge_index[0].astype(jnp.int32)
  dst = edge_index[1].astype(jnp.int32)
  pad = EP - E
  pad_ar = jnp.arange(pad, dtype=jnp.int32)
  src_p = jnp.concatenate([src, (pad_ar * 61) % N])
  dst_p = jnp.concatenate([dst, N + pad_ar % (ACC_ROWS - N)])
  src0 = src_p.reshape(IDX_ROWS, K)
  src1 = src0 + N
  dst2 = dst_p.reshape(IDX_ROWS, K)
  zeros = jnp.zeros((ACC_ROWS, H), jnp.float32)

  wf = _prep(W1, W2[:, :D, :])
  w2b0 = W2[:, D:D + H, :]
  w2b1 = W2[:, D + H:, :]

  for i in range(DEPTH):
    h_stacked = _mm_h(x, W0[i])
    msg = _seg_sum_sc(h_stacked, src0, src1, dst2, zeros)
    x = _combine(x, msg, wf[i], w2b0[i], w2b1[i])
  return x


# speedup vs baseline: 7.2557x; 1.1466x over previous
import functools

import jax
import jax.numpy as jnp
from jax import lax
from jax.experimental import pallas as pl
from jax.experimental.pallas import tpu as pltpu
from jax.experimental.pallas import tpu_sc as plsc

N = 10000
E = 160000
D = 256
DEPTH = 3
H = 128

NC = 2
NS = 16
K = 128
CHUNKS = 80
PIPE = 2
HALVES = 2
EP = NS * CHUNKS * K
IDX_ROWS = EP // K
ACC_ROWS = 10240
ZERO_PER_TILE = ACC_ROWS // NS
OUT_PER_TILE = 1000


def _seg_sum_sc(h_stacked, src0, src1, dst2, zeros):
  mesh = plsc.VectorSubcoreMesh(core_axis_name="c", subcore_axis_name="s",
                                num_cores=NC, num_subcores=NS)

  @functools.partial(
      pl.kernel,
      mesh=mesh,
      out_type=jax.ShapeDtypeStruct((2 * N, H), jnp.float32),
      scratch_types=[
          pltpu.VMEM((CHUNKS // HALVES, K), jnp.int32),
          pltpu.VMEM((CHUNKS // HALVES, K), jnp.int32),
      ] + [pltpu.VMEM((K, H), jnp.float32)] * PIPE
      + [pltpu.VMEM_SHARED((ACC_ROWS, H), jnp.float32)]
      + [pltpu.SemaphoreType.DMA] * (2 * PIPE),
  )
  def k(h_hbm, src0_hbm, src1_hbm, dst_hbm, zeros_hbm, out_hbm,
        src_v, dst_v, *rest):
    rows = rest[:PIPE]
    acc = rest[PIPE]
    gsem = rest[PIPE + 1:2 * PIPE + 1]
    ssem = rest[2 * PIPE + 1:]
    cid = lax.axis_index("c")
    sid = lax.axis_index("s")
    hr = CHUNKS // HALVES

    pltpu.sync_copy(zeros_hbm.at[pl.ds(sid * ZERO_PER_TILE, ZERO_PER_TILE)],
                    acc.at[pl.ds(sid * ZERO_PER_TILE, ZERO_PER_TILE)])
    plsc.subcore_barrier()

    def gather(j, b):
      pltpu.async_copy(h_hbm.at[src_v.at[j]], rows[b], gsem[b])

    def wait_gather(j, b):
      pltpu.make_async_copy(h_hbm.at[src_v.at[j]], rows[b], gsem[b]).wait()

    def scatter(j, b):
      pltpu.async_copy(rows[b], acc.at[dst_v.at[j]], ssem[b], add=True)

    def wait_scatter(j, b):
      pltpu.make_async_copy(rows[b], acc.at[dst_v.at[j]], ssem[b]).wait()

    for half in range(HALVES):
      base = sid * CHUNKS + half * hr
      @pl.when(cid == 0)
      def _():
        pltpu.sync_copy(src0_hbm.at[pl.ds(base, hr)], src_v)

      @pl.when(cid != 0)
      def _():
        pltpu.sync_copy(src1_hbm.at[pl.ds(base, hr)], src_v)

      pltpu.sync_copy(dst_hbm.at[pl.ds(base, hr)], dst_v)

      gather(0, 0)

      def step(j, bcur, bnext):
        @pl.when(j >= 1)
        def _():
          wait_scatter(j - 1, bnext)

        @pl.when(j + 1 < hr)
        def _():
          gather(j + 1, bnext)

        wait_gather(j, bcur)
        scatter(j, bcur)

      def body(t, carry):
        step(2 * t, 0, 1)
        step(2 * t + 1, 1, 0)
        return carry

      lax.fori_loop(0, hr // 2, body, 0)
      wait_scatter(hr - 1, 1)
    plsc.subcore_barrier()

    @pl.when(sid < 10)
    def _():
      pltpu.sync_copy(
          acc.at[pl.ds(sid * OUT_PER_TILE, OUT_PER_TILE)],
          out_hbm.at[pl.ds(cid * N + sid * OUT_PER_TILE, OUT_PER_TILE)])

  return k(h_stacked, src0, src1, dst2, zeros)


RB = 2000
NB = N // RB


def _mm_h_kernel(x_ref, w_ref, o_ref):
  o_ref[...] = jnp.dot(x_ref[...], w_ref[...],
                       preferred_element_type=jnp.float32)


def _mm_h(x, w0):
  return pl.pallas_call(
      _mm_h_kernel,
      grid=(NC, NB),
      in_specs=[
          pl.BlockSpec((RB, D), lambda c, i: (i, 0)),
          pl.BlockSpec((D, H), lambda c, i: (0, c)),
      ],
      out_specs=pl.BlockSpec((RB, H), lambda c, i: (c * NB + i, 0)),
      out_shape=jax.ShapeDtypeStruct((2 * N, H), jnp.float32),
  )(x, w0)


def _combine_kernel(x_ref, ma_ref, mb_ref, wf_ref, wa_ref, wb_ref, o_ref):
  y = jnp.dot(x_ref[...], wf_ref[...], preferred_element_type=jnp.float32)
  y += jnp.dot(ma_ref[...], wa_ref[...], preferred_element_type=jnp.float32)
  y += jnp.dot(mb_ref[...], wb_ref[...], preferred_element_type=jnp.float32)
  mu = jnp.mean(y, axis=1, keepdims=True)
  d = y - mu
  var = jnp.mean(d * d, axis=1, keepdims=True)
  o_ref[...] = y * lax.rsqrt(var)


def _combine(x, msg, wf, w2b0, w2b1):
  return pl.pallas_call(
      _combine_kernel,
      grid=(NB,),
      in_specs=[
          pl.BlockSpec((RB, D), lambda i: (i, 0)),
          pl.BlockSpec((RB, H), lambda i: (i, 0)),
          pl.BlockSpec((RB, H), lambda i: (i + NB, 0)),
          pl.BlockSpec((D, D), lambda i: (0, 0)),
          pl.BlockSpec((H, D), lambda i: (0, 0)),
          pl.BlockSpec((H, D), lambda i: (0, 0)),
      ],
      out_specs=pl.BlockSpec((RB, D), lambda i: (i, 0)),
      out_shape=jax.ShapeDtypeStruct((N, D), jnp.float32),
  )(x, msg, msg, wf, w2b0, w2b1)


def _prep_kernel(w1_ref, w2_ref, o_ref):
  o_ref[0] = jnp.dot(w1_ref[0], w2_ref[0],
                     preferred_element_type=jnp.float32)


def _prep(w1, w2a):
  return pl.pallas_call(
      _prep_kernel,
      grid=(DEPTH,),
      in_specs=[
          pl.BlockSpec((1, D, D), lambda i: (i, 0, 0)),
          pl.BlockSpec((1, D, D), lambda i: (i, 0, 0)),
      ],
      out_specs=pl.BlockSpec((1, D, D), lambda i: (i, 0, 0)),
      out_shape=jax.ShapeDtypeStruct((DEPTH, D, D), jnp.float32),
  )(w1, w2a)


def kernel(x, edge_index, W0, W1, W2):
  src = edge_index[0].astype(jnp.int32)
  dst = edge_index[1].astype(jnp.int32)
  pad = EP - E
  pad_ar = jnp.arange(pad, dtype=jnp.int32)
  src_p = jnp.concatenate([src, (pad_ar * 61) % N])
  dst_p = jnp.concatenate([dst, N + pad_ar % (ACC_ROWS - N)])
  src0 = src_p.reshape(IDX_ROWS, K)
  src1 = src0 + N
  dst2 = dst_p.reshape(IDX_ROWS, K)
  zeros = jnp.zeros((ACC_ROWS, H), jnp.float32)

  wf = _prep(W1, W2[:, :D, :])
  w2b0 = W2[:, D:D + H, :]
  w2b1 = W2[:, D + H:, :]

  for i in range(DEPTH):
    h_stacked = _mm_h(x, W0[i])
    msg = _seg_sum_sc(h_stacked, src0, src1, dst2, zeros)
    x = _combine(x, msg, wf[i], w2b0[i], w2b1[i])
  return x
